# baseline (speedup 1.0000x reference)
import jax
import jax.numpy as jnp
from jax import lax
from jax.experimental import pallas as pl
from jax.experimental.pallas import tpu as pltpu

N = 100000
D = 128
B = 64
R = 2000
G = N // R


def _body(ids_ref, x_ref, w_ref, b_ref, ms_ref, out_ref,
          acc_ref, cnt_ref, tab_ref):
    i = pl.program_id(0)
    ids = ids_ref[0]
    iota = lax.broadcasted_iota(jnp.int32, (B, 1), 0)
    ohf = (iota == ids).astype(jnp.float32)
    oh = ohf.astype(jnp.bfloat16)

    @pl.when(i == 0)
    def _():
        acc_ref[...] = jnp.zeros_like(acc_ref)
        cnt_ref[...] = jnp.zeros_like(cnt_ref)

    @pl.when(i < G)
    def _():
        x = x_ref[...]
        rhs = jnp.concatenate([x, x * x], axis=1).astype(jnp.bfloat16)
        acc_ref[...] += lax.dot_general(
            oh, rhs, (((1,), (0,)), ((), ())),
            preferred_element_type=jnp.float32)
        cnt_ref[...] += jnp.sum(ohf, axis=1, keepdims=True)

    @pl.when(i == G)
    def _():
        c = jnp.maximum(cnt_ref[...], 1.0)
        s = acc_ref[:, :D]
        q = acc_ref[:, D:]
        mean = s / c
        mm = mean * ms_ref[...]
        segsq = q - 2.0 * mm * s + c * mm * mm
        rstd = lax.rsqrt(segsq / c + 1e-6)
        scale = w_ref[...] * rstd
        shift = b_ref[...] - mm * scale
        tab_ref[...] = jnp.concatenate([scale, shift],
                                       axis=1).astype(jnp.bfloat16)

    @pl.when(i >= G)
    def _():
        g = lax.dot_general(oh, tab_ref[...], (((0,), (0,)), ((), ())),
                            preferred_element_type=jnp.float32)
        out_ref[...] = x_ref[...] * g[:, :D] + g[:, D:]


def _graph_norm(x, seg_row, weight, bias, mean_scale):
    return pl.pallas_call(
        _body,
        grid=(2 * G,),
        in_specs=[
            pl.BlockSpec((1, 1, R), lambda i: (i % G, 0, 0)),
            pl.BlockSpec((R, D), lambda i: (i % G, 0)),
            pl.BlockSpec((1, D), lambda i: (0, 0)),
            pl.BlockSpec((1, D), lambda i: (0, 0)),
            pl.BlockSpec((1, D), lambda i: (0, 0)),
        ],
        out_specs=pl.BlockSpec((R, D),
                               lambda i: (jnp.where(i < G, 0, i - G), 0)),
        out_shape=jax.ShapeDtypeStruct((N, D), jnp.float32),
        scratch_shapes=[
            pltpu.VMEM((B, 2 * D), jnp.float32),
            pltpu.VMEM((B, 1), jnp.float32),
            pltpu.VMEM((B, 2 * D), jnp.bfloat16),
        ],
    )(seg_row, x, weight, bias, mean_scale)


def kernel(x, segment_ids, weight, bias, mean_scale):
    seg = segment_ids.astype(jnp.int32)
    seg_row = seg.reshape(G, 1, R)
    w = weight.reshape(1, D)
    b = bias.reshape(1, D)
    ms = mean_scale.reshape(1, D)
    return _graph_norm(x, seg_row, w, b, ms)


# speedup vs baseline: 1.4174x; 1.4174x over previous
import jax
import jax.numpy as jnp
from jax import lax
from jax.experimental import pallas as pl
from jax.experimental.pallas import tpu as pltpu

N = 100000
D = 128
B = 64
R = 4000
G = N // R


def _body(ids_ref, x_ref, w_ref, b_ref, ms_ref, out_ref,
          acc_ref, cnt_ref, tab_ref):
    i = pl.program_id(0)
    ids = ids_ref[0]
    iota = lax.broadcasted_iota(jnp.int32, (B, 1), 0)
    ohf = (iota == ids).astype(jnp.float32)
    oh = ohf.astype(jnp.bfloat16)

    @pl.when(i == 0)
    def _():
        acc_ref[...] = jnp.zeros_like(acc_ref)
        cnt_ref[...] = jnp.zeros_like(cnt_ref)

    @pl.when(i < G)
    def _():
        x = x_ref[...]
        rhs = jnp.concatenate([x, x * x], axis=1).astype(jnp.bfloat16)
        acc_ref[...] += lax.dot_general(
            oh, rhs, (((1,), (0,)), ((), ())),
            preferred_element_type=jnp.float32)
        cnt_ref[...] += jnp.sum(ohf, axis=1, keepdims=True)

    @pl.when(i == G)
    def _():
        c = jnp.maximum(cnt_ref[...], 1.0)
        s = acc_ref[:, :D]
        q = acc_ref[:, D:]
        mean = s / c
        mm = mean * ms_ref[...]
        segsq = q - 2.0 * mm * s + c * mm * mm
        rstd = lax.rsqrt(segsq / c + 1e-6)
        scale = w_ref[...] * rstd
        shift = b_ref[...] - mm * scale
        tab_ref[...] = jnp.concatenate([scale, shift],
                                       axis=1).astype(jnp.bfloat16)

    @pl.when(i >= G)
    def _():
        g = lax.dot_general(oh, tab_ref[...], (((0,), (0,)), ((), ())),
                            preferred_element_type=jnp.float32)
        out_ref[...] = x_ref[...] * g[:, :D] + g[:, D:]


def _graph_norm(x, seg_row, weight, bias, mean_scale):
    return pl.pallas_call(
        _body,
        grid=(2 * G,),
        in_specs=[
            pl.BlockSpec((1, 1, R), lambda i: (i % G, 0, 0)),
            pl.BlockSpec((R, D), lambda i: (i % G, 0)),
            pl.BlockSpec((1, D), lambda i: (0, 0)),
            pl.BlockSpec((1, D), lambda i: (0, 0)),
            pl.BlockSpec((1, D), lambda i: (0, 0)),
        ],
        out_specs=pl.BlockSpec((R, D),
                               lambda i: (jnp.where(i < G, 0, i - G), 0)),
        out_shape=jax.ShapeDtypeStruct((N, D), jnp.float32),
        scratch_shapes=[
            pltpu.VMEM((B, 2 * D), jnp.float32),
            pltpu.VMEM((B, 1), jnp.float32),
            pltpu.VMEM((B, 2 * D), jnp.bfloat16),
        ],
    )(seg_row, x, weight, bias, mean_scale)


def kernel(x, segment_ids, weight, bias, mean_scale):
    seg = segment_ids.astype(jnp.int32)
    seg_row = seg.reshape(G, 1, R)
    w = weight.reshape(1, D)
    b = bias.reshape(1, D)
    ms = mean_scale.reshape(1, D)
    return _graph_norm(x, seg_row, w, b, ms)


# speedup vs baseline: 1.8021x; 1.2714x over previous
import jax
import jax.numpy as jnp
from jax import lax
from jax.experimental import pallas as pl
from jax.experimental.pallas import tpu as pltpu

N = 100000
D = 128
B = 64
R = 10000
G = N // R


def _body(ids_ref, x_ref, w_ref, b_ref, ms_ref, out_ref,
          acc_ref, cnt_ref, tab_ref):
    i = pl.program_id(0)
    ids = ids_ref[0]
    iota = lax.broadcasted_iota(jnp.int32, (B, 1), 0)
    ohf = (iota == ids).astype(jnp.float32)
    oh = ohf.astype(jnp.bfloat16)

    @pl.when(i == 0)
    def _():
        acc_ref[...] = jnp.zeros_like(acc_ref)
        cnt_ref[...] = jnp.zeros_like(cnt_ref)

    @pl.when(i < G)
    def _():
        x = x_ref[...]
        rhs = jnp.concatenate([x, x * x], axis=1).astype(jnp.bfloat16)
        acc_ref[...] += lax.dot_general(
            oh, rhs, (((1,), (0,)), ((), ())),
            preferred_element_type=jnp.float32)
        cnt_ref[...] += jnp.sum(ohf, axis=1, keepdims=True)

    @pl.when(i == G)
    def _():
        c = jnp.maximum(cnt_ref[...], 1.0)
        s = acc_ref[:, :D]
        q = acc_ref[:, D:]
        mean = s / c
        mm = mean * ms_ref[...]
        segsq = q - 2.0 * mm * s + c * mm * mm
        rstd = lax.rsqrt(segsq / c + 1e-6)
        scale = w_ref[...] * rstd
        shift = b_ref[...] - mm * scale
        tab_ref[...] = jnp.concatenate([scale, shift],
                                       axis=1).astype(jnp.bfloat16)

    @pl.when(i >= G)
    def _():
        g = lax.dot_general(oh, tab_ref[...], (((0,), (0,)), ((), ())),
                            preferred_element_type=jnp.float32)
        out_ref[...] = x_ref[...] * g[:, :D] + g[:, D:]


def _graph_norm(x, seg_row, weight, bias, mean_scale):
    return pl.pallas_call(
        _body,
        grid=(2 * G,),
        in_specs=[
            pl.BlockSpec((1, 1, R), lambda i: (i % G, 0, 0)),
            pl.BlockSpec((R, D), lambda i: (i % G, 0)),
            pl.BlockSpec((1, D), lambda i: (0, 0)),
            pl.BlockSpec((1, D), lambda i: (0, 0)),
            pl.BlockSpec((1, D), lambda i: (0, 0)),
        ],
        out_specs=pl.BlockSpec((R, D),
                               lambda i: (jnp.where(i < G, 0, i - G), 0)),
        out_shape=jax.ShapeDtypeStruct((N, D), jnp.float32),
        scratch_shapes=[
            pltpu.VMEM((B, 2 * D), jnp.float32),
            pltpu.VMEM((B, 1), jnp.float32),
            pltpu.VMEM((B, 2 * D), jnp.bfloat16),
        ],
    )(seg_row, x, weight, bias, mean_scale)


def kernel(x, segment_ids, weight, bias, mean_scale):
    seg = segment_ids.astype(jnp.int32)
    seg_row = seg.reshape(G, 1, R)
    w = weight.reshape(1, D)
    b = bias.reshape(1, D)
    ms = mean_scale.reshape(1, D)
    return _graph_norm(x, seg_row, w, b, ms)


# speedup vs baseline: 1.9191x; 1.0649x over previous
import jax
import jax.numpy as jnp
from jax import lax
from jax.experimental import pallas as pl
from jax.experimental.pallas import tpu as pltpu

N = 100000
D = 128
B = 64
R = 20000
G = N // R


def _body(ids_ref, x_ref, w_ref, b_ref, ms_ref, out_ref,
          acc_ref, cnt_ref, tab_ref):
    i = pl.program_id(0)
    ids = ids_ref[0]
    iota = lax.broadcasted_iota(jnp.int32, (B, 1), 0)
    ohf = (iota == ids).astype(jnp.float32)
    oh = ohf.astype(jnp.bfloat16)

    @pl.when(i == 0)
    def _():
        acc_ref[...] = jnp.zeros_like(acc_ref)
        cnt_ref[...] = jnp.zeros_like(cnt_ref)

    @pl.when(i < G)
    def _():
        x = x_ref[...]
        rhs = jnp.concatenate([x, x * x], axis=1).astype(jnp.bfloat16)
        acc_ref[...] += lax.dot_general(
            oh, rhs, (((1,), (0,)), ((), ())),
            preferred_element_type=jnp.float32)
        cnt_ref[...] += jnp.sum(ohf, axis=1, keepdims=True)

    @pl.when(i == G)
    def _():
        c = jnp.maximum(cnt_ref[...], 1.0)
        s = acc_ref[:, :D]
        q = acc_ref[:, D:]
        mean = s / c
        mm = mean * ms_ref[...]
        segsq = q - 2.0 * mm * s + c * mm * mm
        rstd = lax.rsqrt(segsq / c + 1e-6)
        scale = w_ref[...] * rstd
        shift = b_ref[...] - mm * scale
        tab_ref[...] = jnp.concatenate([scale, shift],
                                       axis=1).astype(jnp.bfloat16)

    @pl.when(i >= G)
    def _():
        g = lax.dot_general(oh, tab_ref[...], (((0,), (0,)), ((), ())),
                            preferred_element_type=jnp.float32)
        out_ref[...] = x_ref[...] * g[:, :D] + g[:, D:]


def _graph_norm(x, seg_row, weight, bias, mean_scale):
    return pl.pallas_call(
        _body,
        grid=(2 * G,),
        in_specs=[
            pl.BlockSpec((1, 1, R), lambda i: (i % G, 0, 0)),
            pl.BlockSpec((R, D), lambda i: (i % G, 0)),
            pl.BlockSpec((1, D), lambda i: (0, 0)),
            pl.BlockSpec((1, D), lambda i: (0, 0)),
            pl.BlockSpec((1, D), lambda i: (0, 0)),
        ],
        out_specs=pl.BlockSpec((R, D),
                               lambda i: (jnp.where(i < G, 0, i - G), 0)),
        out_shape=jax.ShapeDtypeStruct((N, D), jnp.float32),
        scratch_shapes=[
            pltpu.VMEM((B, 2 * D), jnp.float32),
            pltpu.VMEM((B, 1), jnp.float32),
            pltpu.VMEM((B, 2 * D), jnp.bfloat16),
        ],
    )(seg_row, x, weight, bias, mean_scale)


def kernel(x, segment_ids, weight, bias, mean_scale):
    seg = segment_ids.astype(jnp.int32)
    seg_row = seg.reshape(G, 1, R)
    w = weight.reshape(1, D)
    b = bias.reshape(1, D)
    ms = mean_scale.reshape(1, D)
    return _graph_norm(x, seg_row, w, b, ms)


# speedup vs baseline: 2.0469x; 1.0666x over previous
import jax
import jax.numpy as jnp
from jax import lax
from jax.experimental import pallas as pl
from jax.experimental.pallas import tpu as pltpu

N = 100000
D = 128
B = 64
R = 10000
G = N // R
K = 8


def _body(ids_ref, x_ref, w_ref, b_ref, ms_ref, out_ref,
          acc_ref, cnt_ref, tab_ref, cache_ref):
    i = pl.program_id(0)
    ids = ids_ref[0]
    iota = lax.broadcasted_iota(jnp.int32, (B, 1), 0)
    ohf = (iota == ids).astype(jnp.float32)
    oh = ohf.astype(jnp.bfloat16)

    @pl.when(i == 0)
    def _():
        acc_ref[...] = jnp.zeros_like(acc_ref)
        cnt_ref[...] = jnp.zeros_like(cnt_ref)

    @pl.when(i < G)
    def _():
        x = x_ref[...]
        xb = x.astype(jnp.bfloat16)
        rhs = jnp.concatenate([xb, (x * x).astype(jnp.bfloat16)], axis=1)
        acc_ref[...] += lax.dot_general(
            oh, rhs, (((1,), (0,)), ((), ())),
            preferred_element_type=jnp.float32)
        cnt_ref[...] += jnp.sum(ohf, axis=1, keepdims=True)

        @pl.when(i < K)
        def _():
            cache_ref[pl.ds(i * R, R), :] = xb

    @pl.when(i == G)
    def _():
        c = jnp.maximum(cnt_ref[...], 1.0)
        s = acc_ref[:, :D]
        q = acc_ref[:, D:]
        mean = s / c
        mm = mean * ms_ref[...]
        segsq = q - 2.0 * mm * s + c * mm * mm
        rstd = lax.rsqrt(segsq / c + 1e-6)
        scale = w_ref[...] * rstd
        shift = b_ref[...] - mm * scale
        tab_ref[...] = jnp.concatenate([scale, shift],
                                       axis=1).astype(jnp.bfloat16)

    def _apply(xa):
        g = lax.dot_general(oh, tab_ref[...], (((0,), (0,)), ((), ())),
                            preferred_element_type=jnp.float32)
        out_ref[...] = xa * g[:, :D] + g[:, D:]

    @pl.when((i >= G) & (i < G + K))
    def _():
        _apply(cache_ref[pl.ds((i - G) * R, R), :].astype(jnp.float32))

    @pl.when(i >= G + K)
    def _():
        _apply(x_ref[...])


def _x_index(i):
    j = i - G
    return (jnp.where(i < G, i, jnp.where(j < K, G - 1, j)), 0)


def _graph_norm(x, seg_row, weight, bias, mean_scale):
    return pl.pallas_call(
        _body,
        grid=(2 * G,),
        in_specs=[
            pl.BlockSpec((1, 1, R), lambda i: (i % G, 0, 0)),
            pl.BlockSpec((R, D), _x_index),
            pl.BlockSpec((1, D), lambda i: (0, 0)),
            pl.BlockSpec((1, D), lambda i: (0, 0)),
            pl.BlockSpec((1, D), lambda i: (0, 0)),
        ],
        out_specs=pl.BlockSpec((R, D),
                               lambda i: (jnp.where(i < G, 0, i - G), 0)),
        out_shape=jax.ShapeDtypeStruct((N, D), jnp.float32),
        scratch_shapes=[
            pltpu.VMEM((B, 2 * D), jnp.float32),
            pltpu.VMEM((B, 1), jnp.float32),
            pltpu.VMEM((B, 2 * D), jnp.bfloat16),
            pltpu.VMEM((K * R, D), jnp.bfloat16),
        ],
    )(seg_row, x, weight, bias, mean_scale)


def kernel(x, segment_ids, weight, bias, mean_scale):
    seg = segment_ids.astype(jnp.int32)
    seg_row = seg.reshape(G, 1, R)
    w = weight.reshape(1, D)
    b = bias.reshape(1, D)
    ms = mean_scale.reshape(1, D)
    return _graph_norm(x, seg_row, w, b, ms)
